# baseline (speedup 1.0000x reference)
import functools

import jax
import jax.numpy as jnp
from jax import lax
from jax.experimental import pallas as pl
from jax.experimental.pallas import tpu as pltpu
from jax.experimental.pallas import tpu_sc as plsc

_N = 10000
_E = 320000
_NC, _NS = 2, 16
_W = 128

_EB = 1600
_NEB = _E // _EB


def _mesh():
    return plsc.VectorSubcoreMesh(
        core_axis_name="c", subcore_axis_name="s",
        num_cores=_NC, num_subcores=_NS)


_GB = 4


def _sc_gather(table, idx):
    b = idx.shape[0]
    d = table.shape[1]
    idx2 = idx.reshape(1, b)
    nblk = b // _W

    @functools.partial(
        pl.kernel,
        out_type=jax.ShapeDtypeStruct((b, d), table.dtype),
        mesh=_mesh(),
        compiler_params=pltpu.CompilerParams(use_tc_tiling_on_sc=False),
    )
    def k(t_hbm, i_hbm, o_hbm):
        def body(i_vmem, o_vmem):
            pltpu.sync_copy(t_hbm.at[i_vmem.at[0]], o_vmem)

        pltpu.emit_pipeline(
            body,
            grid=(nblk,),
            in_specs=[pl.BlockSpec((1, _W), lambda i: (0, i))],
            out_specs=[pl.BlockSpec((_W, d), lambda i: (i, 0))],
            core_axis_name=("c", "s"),
            dimension_semantics=(pltpu.PARALLEL,),
        )(i_hbm, o_hbm)

    return k(table, idx2)


def _sc_scatter_add(vals, idx, n_rows):
    e, d = vals.shape
    idx2 = idx.reshape(1, e)
    nblk = e // _W
    rps = n_rows // _NS
    zeros = jnp.zeros((n_rows, d), vals.dtype)

    @functools.partial(
        pl.kernel,
        out_type=jax.ShapeDtypeStruct((_NC, n_rows, d), vals.dtype),
        mesh=_mesh(),
        scratch_types=[pltpu.VMEM_SHARED((n_rows, d), vals.dtype)],
        compiler_params=pltpu.CompilerParams(use_tc_tiling_on_sc=False),
    )
    def k(v_hbm, i_hbm, z_hbm, o_hbm, acc):
        c = lax.axis_index("c")
        s = lax.axis_index("s")
        slab = pl.ds(s * rps, rps)
        pltpu.sync_copy(z_hbm.at[slab], acc.at[slab])
        plsc.subcore_barrier()

        def body(v_vmem, i_vmem):
            pltpu.sync_copy(v_vmem, acc.at[i_vmem.at[0]], add=True)

        pltpu.emit_pipeline(
            body,
            grid=(nblk,),
            in_specs=[pl.BlockSpec((_W, d), lambda i: (i, 0)),
                      pl.BlockSpec((1, _W), lambda i: (0, i))],
            out_specs=[],
            core_axis_name=("c", "s"),
            dimension_semantics=(pltpu.PARALLEL,),
        )(v_hbm, i_hbm)

        plsc.subcore_barrier()
        pltpu.sync_copy(acc.at[slab], o_hbm.at[c].at[slab])

    return k(vals, idx2, zeros)


def _tc_mlp2(xin, ps, blk, out_dtype=jnp.float32):
    (w1, b1), (w2, b2) = ps
    n = xin.shape[0]
    d_out = w2.shape[1]
    grid = n // blk
    b1r = b1.reshape(1, -1)
    b2r = b2.reshape(1, -1)

    def body(x_ref, w1_ref, b1_ref, w2_ref, b2_ref, o_ref):
        h = jnp.dot(x_ref[...], w1_ref[...], preferred_element_type=jnp.float32)
        h = jnp.maximum(h + b1_ref[...], 0.0)
        o = jnp.dot(h, w2_ref[...], preferred_element_type=jnp.float32)
        o_ref[...] = jnp.maximum(o + b2_ref[...], 0.0).astype(out_dtype)

    full = lambda a: pl.BlockSpec(a.shape, lambda i: (0, 0))
    return pl.pallas_call(
        body,
        grid=(grid,),
        in_specs=[
            pl.BlockSpec((blk, xin.shape[1]), lambda i: (i, 0)),
            full(w1), full(b1r), full(w2), full(b2r),
        ],
        out_specs=pl.BlockSpec((blk, d_out), lambda i: (i, 0)),
        out_shape=jax.ShapeDtypeStruct((n, d_out), out_dtype),
    )(xin, w1, b1r, w2, b2r)


def _tc_dense_step(g, lat_e, init_e, row3, col3, wp):
    (w1, b1, w2, b2, wio1, bio1, wio2, bio2, wc1, bc1, wc2, bc2) = wp

    def body(er, ec, gr, gc, le, ie,
             w1_, b1_, w2_, b2_, wio1_, bio1_, wio2_, bio2_,
             wc1_, bc1_, wc2_, bc2_,
             le_o, fc_o, cls_o):
        f32 = jnp.float32
        gc_v = gc[...].astype(f32)
        le_v = le[...]
        xcat = jnp.concatenate([gr[...].astype(f32), gc_v, ie[...], le_v], axis=1)
        h = jnp.dot(xcat, w1_[...], preferred_element_type=f32)
        h = jnp.maximum(h + b1_[...], 0.0)
        le_n = jnp.dot(h, w2_[...], preferred_element_type=f32)
        le_n = jnp.maximum(le_n + b2_[...], 0.0)
        le_o[...] = le_n

        nbr = jnp.concatenate([gc_v, le_n], axis=1)
        hio = jnp.dot(nbr, wio1_[...], preferred_element_type=f32)
        hio = jnp.maximum(hio + bio1_[...], 0.0)
        f = jnp.dot(hio, wio2_[...], preferred_element_type=f32)
        f = jnp.maximum(f + bio2_[...], 0.0)

        rows = er[0, 0, :]
        cols = ec[0, 0, :]
        mi = (rows > cols).astype(f32)[:, None]
        mo = (rows < cols).astype(f32)[:, None]
        half = jax.lax.broadcasted_iota(jnp.int32, f.shape, 1) < 32
        fc_o[...] = f * jnp.where(half, mi, mo)

        hc = jnp.maximum(
            jnp.dot(le_n, wc1_[...], preferred_element_type=f32) + bc1_[...], 0.0)
        cls_o[...] = jnp.dot(hc, wc2_[...], preferred_element_type=f32) + bc2_[...]

    full = lambda a: pl.BlockSpec(a.shape, lambda i: (0, 0))
    wspecs = [full(w) for w in wp]
    return pl.pallas_call(
        body,
        grid=(_NEB,),
        in_specs=[
            pl.BlockSpec((1, 1, _EB), lambda i: (i, 0, 0)),
            pl.BlockSpec((1, 1, _EB), lambda i: (i, 0, 0)),
            pl.BlockSpec((_EB, 32), lambda i: (i, 0)),
            pl.BlockSpec((_EB, 32), lambda i: (i + _NEB, 0)),
            pl.BlockSpec((_EB, 16), lambda i: (i, 0)),
            pl.BlockSpec((_EB, 16), lambda i: (i, 0)),
        ] + wspecs,
        out_specs=[
            pl.BlockSpec((_EB, 16), lambda i: (i, 0)),
            pl.BlockSpec((_EB, 64), lambda i: (i, 0)),
            pl.BlockSpec((_EB, 1), lambda i: (i, 0)),
        ],
        out_shape=[
            jax.ShapeDtypeStruct((_E, 16), jnp.float32),
            jax.ShapeDtypeStruct((_E, 64), jnp.float32),
            jax.ShapeDtypeStruct((_E, 1), jnp.float32),
        ],
    )(row3, col3, g, g, lat_e, init_e, *wp)


def _tc_node_update(partials, ps):
    (w, b) = ps[0]
    br = b.reshape(1, -1)

    def body(p_ref, w_ref, b_ref, o_ref):
        flow = p_ref[0] + p_ref[1]
        o = jnp.dot(flow, w_ref[...], preferred_element_type=jnp.float32)
        o_ref[...] = jnp.maximum(o + b_ref[...], 0.0).astype(jnp.bfloat16)

    full = lambda a: pl.BlockSpec(a.shape, lambda i: tuple(0 for _ in a.shape))
    return pl.pallas_call(
        body,
        grid=(1,),
        in_specs=[full(partials), full(w), full(br)],
        out_specs=pl.BlockSpec((_N, w.shape[1]), lambda i: (0, 0)),
        out_shape=jax.ShapeDtypeStruct((_N, w.shape[1]), jnp.bfloat16),
    )(partials, w, br)


def _split_step_weights(params):
    (w1, b1), (w2, b2) = params['edge_model']
    (wi1, bi1), (wi2, bi2) = params['flow_in']
    (wo1, bo1), (wo2, bo2) = params['flow_out']
    (wc1, bc1), (wc2, bc2) = params['cls_edge']
    wio1 = jnp.concatenate([wi1, wo1], axis=1)
    bio1 = jnp.concatenate([bi1, bo1]).reshape(1, -1)
    wio2 = jnp.zeros((112, 64), jnp.float32)
    wio2 = wio2.at[:56, :32].set(wi2).at[56:, 32:].set(wo2)
    bio2 = jnp.concatenate([bi2, bo2]).reshape(1, -1)
    return (
        w1, b1.reshape(1, -1), w2, b2.reshape(1, -1),
        wio1, bio1, wio2, bio2,
        wc1, bc1.reshape(1, -1), wc2, bc2.reshape(1, -1),
    )


def kernel(x, edge_index, edge_attr, params):
    lat_n = _tc_mlp2(x, params['enc_node'], blk=2000, out_dtype=jnp.bfloat16)
    lat_e = _tc_mlp2(edge_attr, params['enc_edge'], blk=8000)
    init_e = lat_e
    wp = _split_step_weights(params)
    gid = edge_index.reshape(-1)
    row = edge_index[0]
    row3 = row.reshape(_NEB, 1, _EB)
    col3 = edge_index[1].reshape(_NEB, 1, _EB)
    outs = []
    for step in range(1, 5):
        g = _sc_gather(lat_n, gid)
        lat_e, f_cat, cls = _tc_dense_step(g, lat_e, init_e, row3, col3, wp)
        partials = _sc_scatter_add(f_cat, row, _N)
        lat_n = _tc_node_update(partials, params['node_update'])
        if step >= 2:
            outs.append(cls)
    return jnp.stack(outs)


# speedup vs baseline: 1.0280x; 1.0280x over previous
import functools

import jax
import jax.numpy as jnp
from jax import lax
from jax.experimental import pallas as pl
from jax.experimental.pallas import tpu as pltpu
from jax.experimental.pallas import tpu_sc as plsc

_N = 10000
_E = 320000
_NC, _NS = 2, 16
_W = 128

_EB = 1600
_NEB = _E // _EB


def _mesh():
    return plsc.VectorSubcoreMesh(
        core_axis_name="c", subcore_axis_name="s",
        num_cores=_NC, num_subcores=_NS)


_GB = 4


def _sc_gather(table, idx):
    b = idx.shape[0]
    d = table.shape[1]
    idx2 = idx.reshape(1, b)
    nblk = b // _W

    @functools.partial(
        pl.kernel,
        out_type=jax.ShapeDtypeStruct((b, d), table.dtype),
        mesh=_mesh(),
        compiler_params=pltpu.CompilerParams(use_tc_tiling_on_sc=False),
    )
    def k(t_hbm, i_hbm, o_hbm):
        def body(i_vmem, o_vmem):
            pltpu.sync_copy(t_hbm.at[i_vmem.at[0]], o_vmem)

        pltpu.emit_pipeline(
            body,
            grid=(nblk,),
            in_specs=[pl.BlockSpec((1, _W), lambda i: (0, i))],
            out_specs=[pl.BlockSpec((_W, d), lambda i: (i, 0))],
            core_axis_name=("c", "s"),
            dimension_semantics=(pltpu.PARALLEL,),
        )(i_hbm, o_hbm)

    return k(table, idx2)


def _sc_scatter_add(vals, idx, n_rows):
    e, d = vals.shape
    idx2 = idx.reshape(1, e)
    nblk = e // _W
    rps = n_rows // _NS
    zeros = jnp.zeros((n_rows, d), vals.dtype)

    @functools.partial(
        pl.kernel,
        out_type=jax.ShapeDtypeStruct((_NC, n_rows, d), vals.dtype),
        mesh=_mesh(),
        scratch_types=[pltpu.VMEM_SHARED((n_rows, d), vals.dtype)],
        compiler_params=pltpu.CompilerParams(use_tc_tiling_on_sc=False),
    )
    def k(v_hbm, i_hbm, z_hbm, o_hbm, acc):
        c = lax.axis_index("c")
        s = lax.axis_index("s")
        slab = pl.ds(s * rps, rps)
        pltpu.sync_copy(z_hbm.at[slab], acc.at[slab])
        plsc.subcore_barrier()

        def body(v_vmem, i_vmem):
            pltpu.sync_copy(v_vmem, acc.at[i_vmem.at[0]], add=True)

        pltpu.emit_pipeline(
            body,
            grid=(nblk,),
            in_specs=[pl.BlockSpec((_W, d), lambda i: (i, 0)),
                      pl.BlockSpec((1, _W), lambda i: (0, i))],
            out_specs=[],
            core_axis_name=("c", "s"),
            dimension_semantics=(pltpu.PARALLEL,),
        )(v_hbm, i_hbm)

        plsc.subcore_barrier()
        pltpu.sync_copy(acc.at[slab], o_hbm.at[c].at[slab])

    return k(vals, idx2, zeros)


def _tc_mlp2(xin, ps, blk, out_dtype=jnp.float32):
    (w1, b1), (w2, b2) = ps
    n = xin.shape[0]
    d_out = w2.shape[1]
    grid = n // blk
    b1r = b1.reshape(1, -1)
    b2r = b2.reshape(1, -1)

    def body(x_ref, w1_ref, b1_ref, w2_ref, b2_ref, o_ref):
        h = jnp.dot(x_ref[...], w1_ref[...], preferred_element_type=jnp.float32)
        h = jnp.maximum(h + b1_ref[...], 0.0)
        o = jnp.dot(h, w2_ref[...], preferred_element_type=jnp.float32)
        o_ref[...] = jnp.maximum(o + b2_ref[...], 0.0).astype(out_dtype)

    full = lambda a: pl.BlockSpec(a.shape, lambda i: (0, 0))
    return pl.pallas_call(
        body,
        grid=(grid,),
        in_specs=[
            pl.BlockSpec((blk, xin.shape[1]), lambda i: (i, 0)),
            full(w1), full(b1r), full(w2), full(b2r),
        ],
        out_specs=pl.BlockSpec((blk, d_out), lambda i: (i, 0)),
        out_shape=jax.ShapeDtypeStruct((n, d_out), out_dtype),
    )(xin, w1, b1r, w2, b2r)


def _tc_dense_step(g, lat_e, init_e, row3, col3, wp):
    (w1, b1, w2, b2, wio1, bio1, wio2, bio2, wc1, bc1, wc2, bc2) = wp
    ne = lat_e.shape[0]
    neb = ne // _EB

    def body(er, ec, gr, gc, le, ie,
             w1_, b1_, w2_, b2_, wio1_, bio1_, wio2_, bio2_,
             wc1_, bc1_, wc2_, bc2_,
             le_o, fc_o, cls_o):
        f32 = jnp.float32
        gc_v = gc[...]
        le_v = le[...]
        xcat = jnp.concatenate([gr[...], gc_v, ie[...], le_v], axis=1)
        h = jnp.dot(xcat, w1_[...], preferred_element_type=f32)
        h = jnp.maximum(h + b1_[...], 0.0)
        le_n = jnp.dot(h, w2_[...], preferred_element_type=f32)
        le_n = jnp.maximum(le_n + b2_[...], 0.0)
        le_o[...] = le_n

        nbr = jnp.concatenate([gc_v, le_n], axis=1)
        hio = jnp.dot(nbr, wio1_[...], preferred_element_type=f32)
        hio = jnp.maximum(hio + bio1_[...], 0.0)
        f = jnp.dot(hio, wio2_[...], preferred_element_type=f32)
        f = jnp.maximum(f + bio2_[...], 0.0)

        rows = er[0, 0, :]
        cols = ec[0, 0, :]
        mi = (rows > cols).astype(f32)[:, None]
        mo = (rows < cols).astype(f32)[:, None]
        half = jax.lax.broadcasted_iota(jnp.int32, f.shape, 1) < 32
        fc_o[...] = f * jnp.where(half, mi, mo)

        hc = jnp.maximum(
            jnp.dot(le_n, wc1_[...], preferred_element_type=f32) + bc1_[...], 0.0)
        cls_o[...] = jnp.dot(hc, wc2_[...], preferred_element_type=f32) + bc2_[...]

    full = lambda a: pl.BlockSpec(a.shape, lambda i: (0, 0))
    wspecs = [full(w) for w in wp]
    return pl.pallas_call(
        body,
        grid=(neb,),
        in_specs=[
            pl.BlockSpec((1, 1, _EB), lambda i: (i, 0, 0)),
            pl.BlockSpec((1, 1, _EB), lambda i: (i, 0, 0)),
            pl.BlockSpec((_EB, 32), lambda i: (i, 0)),
            pl.BlockSpec((_EB, 32), lambda i: (i + neb, 0)),
            pl.BlockSpec((_EB, 16), lambda i: (i, 0)),
            pl.BlockSpec((_EB, 16), lambda i: (i, 0)),
        ] + wspecs,
        out_specs=[
            pl.BlockSpec((_EB, 16), lambda i: (i, 0)),
            pl.BlockSpec((_EB, 64), lambda i: (i, 0)),
            pl.BlockSpec((_EB, 1), lambda i: (i, 0)),
        ],
        out_shape=[
            jax.ShapeDtypeStruct((ne, 16), jnp.float32),
            jax.ShapeDtypeStruct((ne, 64), jnp.float32),
            jax.ShapeDtypeStruct((ne, 1), jnp.float32),
        ],
    )(row3, col3, g, g, lat_e, init_e, *wp)


def _tc_node_update(partials_list, ps):
    (w, b) = ps[0]
    br = b.reshape(1, -1)

    def body(*refs):
        *p_refs, w_ref, b_ref, o_ref = refs
        flow = sum(p[0] + p[1] for p in p_refs)
        o = jnp.dot(flow, w_ref[...], preferred_element_type=jnp.float32)
        o_ref[...] = jnp.maximum(o + b_ref[...], 0.0)

    full = lambda a: pl.BlockSpec(a.shape, lambda i: tuple(0 for _ in a.shape))
    return pl.pallas_call(
        body,
        grid=(1,),
        in_specs=[full(p) for p in partials_list] + [full(w), full(br)],
        out_specs=pl.BlockSpec((_N, w.shape[1]), lambda i: (0, 0)),
        out_shape=jax.ShapeDtypeStruct((_N, w.shape[1]), jnp.float32),
    )(*partials_list, w, br)


def _split_step_weights(params):
    (w1, b1), (w2, b2) = params['edge_model']
    (wi1, bi1), (wi2, bi2) = params['flow_in']
    (wo1, bo1), (wo2, bo2) = params['flow_out']
    (wc1, bc1), (wc2, bc2) = params['cls_edge']
    wio1 = jnp.concatenate([wi1, wo1], axis=1)
    bio1 = jnp.concatenate([bi1, bo1]).reshape(1, -1)
    wio2 = jnp.zeros((112, 64), jnp.float32)
    wio2 = wio2.at[:56, :32].set(wi2).at[56:, 32:].set(wo2)
    bio2 = jnp.concatenate([bi2, bo2]).reshape(1, -1)
    return (
        w1, b1.reshape(1, -1), w2, b2.reshape(1, -1),
        wio1, bio1, wio2, bio2,
        wc1, bc1.reshape(1, -1), wc2, bc2.reshape(1, -1),
    )


def kernel(x, edge_index, edge_attr, params):
    lat_n = _tc_mlp2(x, params['enc_node'], blk=2000)
    lat_e_full = _tc_mlp2(edge_attr, params['enc_edge'], blk=8000)
    wp = _split_step_weights(params)

    h = _E // 2
    nhb = h // _EB
    rows_ = edge_index[0]
    cols_ = edge_index[1]
    gids = [jnp.concatenate([rows_[:h], cols_[:h]]),
            jnp.concatenate([rows_[h:], cols_[h:]])]
    rowh = [rows_[:h], rows_[h:]]
    row3 = [r.reshape(nhb, 1, _EB) for r in rowh]
    col3 = [c.reshape(nhb, 1, _EB) for c in (cols_[:h], cols_[h:])]
    lat_e = [lat_e_full[:h], lat_e_full[h:]]
    init_e = list(lat_e)

    outs = []
    for step in range(1, 5):
        partials = []
        clses = []
        for half in range(2):
            g = _sc_gather(lat_n, gids[half])
            le_n, f_cat, cls = _tc_dense_step(
                g, lat_e[half], init_e[half], row3[half], col3[half], wp)
            lat_e[half] = le_n
            clses.append(cls)
            partials.append(_sc_scatter_add(f_cat, rowh[half], _N))
        lat_n = _tc_node_update(partials, params['node_update'])
        if step >= 2:
            outs.append(jnp.concatenate(clses))
    return jnp.stack(outs)


# speedup vs baseline: 1.0360x; 1.0078x over previous
import functools

import jax
import jax.numpy as jnp
from jax import lax
from jax.experimental import pallas as pl
from jax.experimental.pallas import tpu as pltpu
from jax.experimental.pallas import tpu_sc as plsc

_N = 10000
_E = 320000
_NC, _NS = 2, 16
_W = 128

_EB = 1600
_NEB = _E // _EB


def _mesh():
    return plsc.VectorSubcoreMesh(
        core_axis_name="c", subcore_axis_name="s",
        num_cores=_NC, num_subcores=_NS)


_GB = 4


def _sc_gather(table, idx):
    b = idx.shape[0]
    d = table.shape[1]
    idx2 = idx.reshape(1, b)
    nblk = b // _W

    @functools.partial(
        pl.kernel,
        out_type=jax.ShapeDtypeStruct((b, d), table.dtype),
        mesh=_mesh(),
        compiler_params=pltpu.CompilerParams(use_tc_tiling_on_sc=False),
    )
    def k(t_hbm, i_hbm, o_hbm):
        def body(i_vmem, o_vmem):
            pltpu.sync_copy(t_hbm.at[i_vmem.at[0]], o_vmem)

        pltpu.emit_pipeline(
            body,
            grid=(nblk,),
            in_specs=[pl.BlockSpec((1, _W), lambda i: (0, i))],
            out_specs=[pl.BlockSpec((_W, d), lambda i: (i, 0))],
            core_axis_name=("c", "s"),
            dimension_semantics=(pltpu.PARALLEL,),
        )(i_hbm, o_hbm)

    return k(table, idx2)


def _sc_scatter_add(vals, idx, n_rows):
    e, d = vals.shape
    idx2 = idx.reshape(1, e)
    nblk = e // _W
    rps = n_rows // _NS
    zeros = jnp.zeros((n_rows, d), vals.dtype)

    @functools.partial(
        pl.kernel,
        out_type=jax.ShapeDtypeStruct((_NC, n_rows, d), vals.dtype),
        mesh=_mesh(),
        scratch_types=[pltpu.VMEM_SHARED((n_rows, d), vals.dtype)],
        compiler_params=pltpu.CompilerParams(use_tc_tiling_on_sc=False),
    )
    def k(v_hbm, i_hbm, z_hbm, o_hbm, acc):
        c = lax.axis_index("c")
        s = lax.axis_index("s")
        slab = pl.ds(s * rps, rps)
        pltpu.sync_copy(z_hbm.at[slab], acc.at[slab])
        plsc.subcore_barrier()

        def body(v_vmem, i_vmem):
            pltpu.sync_copy(v_vmem, acc.at[i_vmem.at[0]], add=True)

        pltpu.emit_pipeline(
            body,
            grid=(nblk,),
            in_specs=[pl.BlockSpec((_W, d), lambda i: (i, 0)),
                      pl.BlockSpec((1, _W), lambda i: (0, i))],
            out_specs=[],
            core_axis_name=("c", "s"),
            dimension_semantics=(pltpu.PARALLEL,),
        )(v_hbm, i_hbm)

        plsc.subcore_barrier()
        pltpu.sync_copy(acc.at[slab], o_hbm.at[c].at[slab])

    return k(vals, idx2, zeros)


def _tc_mlp2(xin, ps, blk, out_dtype=jnp.float32):
    (w1, b1), (w2, b2) = ps
    n = xin.shape[0]
    d_out = w2.shape[1]
    grid = n // blk
    b1r = b1.reshape(1, -1)
    b2r = b2.reshape(1, -1)

    def body(x_ref, w1_ref, b1_ref, w2_ref, b2_ref, o_ref):
        h = jnp.dot(x_ref[...], w1_ref[...], preferred_element_type=jnp.float32)
        h = jnp.maximum(h + b1_ref[...], 0.0)
        o = jnp.dot(h, w2_ref[...], preferred_element_type=jnp.float32)
        o_ref[...] = jnp.maximum(o + b2_ref[...], 0.0).astype(out_dtype)

    full = lambda a: pl.BlockSpec(a.shape, lambda i: (0, 0))
    return pl.pallas_call(
        body,
        grid=(grid,),
        in_specs=[
            pl.BlockSpec((blk, xin.shape[1]), lambda i: (i, 0)),
            full(w1), full(b1r), full(w2), full(b2r),
        ],
        out_specs=pl.BlockSpec((blk, d_out), lambda i: (i, 0)),
        out_shape=jax.ShapeDtypeStruct((n, d_out), out_dtype),
    )(xin, w1, b1r, w2, b2r)


def _tc_dense_step(g, lat_e, init_e, row3, col3, wp):
    (w1, b1, w2, b2, wio1, bio1, wio2, bio2, wc1, bc1, wc2, bc2) = wp
    ne = lat_e.shape[0]
    neb = ne // _EB

    def body(er, ec, gr, gc, le, ie,
             w1_, b1_, w2_, b2_, wio1_, bio1_, wio2_, bio2_,
             wc1_, bc1_, wc2_, bc2_,
             le_o, fc_o, cls_o):
        f32 = jnp.float32
        gc_v = gc[...]
        le_v = le[...]
        xcat = jnp.concatenate([gr[...], gc_v, ie[...], le_v], axis=1)
        h = jnp.dot(xcat, w1_[...], preferred_element_type=f32)
        h = jnp.maximum(h + b1_[...], 0.0)
        le_n = jnp.dot(h, w2_[...], preferred_element_type=f32)
        le_n = jnp.maximum(le_n + b2_[...], 0.0)
        le_o[...] = le_n

        nbr = jnp.concatenate([gc_v, le_n], axis=1)
        hio = jnp.dot(nbr, wio1_[...], preferred_element_type=f32)
        hio = jnp.maximum(hio + bio1_[...], 0.0)
        f = jnp.dot(hio, wio2_[...], preferred_element_type=f32)
        f = jnp.maximum(f + bio2_[...], 0.0)

        rows = er[0, 0, :]
        cols = ec[0, 0, :]
        mi = (rows > cols).astype(f32)[:, None]
        mo = (rows < cols).astype(f32)[:, None]
        fc_o[...] = f[:, :32] * mi + f[:, 32:] * mo

        hc = jnp.maximum(
            jnp.dot(le_n, wc1_[...], preferred_element_type=f32) + bc1_[...], 0.0)
        cls_o[...] = jnp.dot(hc, wc2_[...], preferred_element_type=f32) + bc2_[...]

    full = lambda a: pl.BlockSpec(a.shape, lambda i: (0, 0))
    wspecs = [full(w) for w in wp]
    return pl.pallas_call(
        body,
        grid=(neb,),
        in_specs=[
            pl.BlockSpec((1, 1, _EB), lambda i: (i, 0, 0)),
            pl.BlockSpec((1, 1, _EB), lambda i: (i, 0, 0)),
            pl.BlockSpec((_EB, 32), lambda i: (i, 0)),
            pl.BlockSpec((_EB, 32), lambda i: (i + neb, 0)),
            pl.BlockSpec((_EB, 16), lambda i: (i, 0)),
            pl.BlockSpec((_EB, 16), lambda i: (i, 0)),
        ] + wspecs,
        out_specs=[
            pl.BlockSpec((_EB, 16), lambda i: (i, 0)),
            pl.BlockSpec((_EB, 32), lambda i: (i, 0)),
            pl.BlockSpec((_EB, 1), lambda i: (i, 0)),
        ],
        out_shape=[
            jax.ShapeDtypeStruct((ne, 16), jnp.float32),
            jax.ShapeDtypeStruct((ne, 32), jnp.float32),
            jax.ShapeDtypeStruct((ne, 1), jnp.float32),
        ],
    )(row3, col3, g, g, lat_e, init_e, *wp)


def _tc_node_update(partials_list, ps):
    (w, b) = ps[0]
    w_in, w_out = w[:32], w[32:]
    br = b.reshape(1, -1)

    def body(*refs):
        *p_refs, wi_ref, wo_ref, b_ref, o_ref = refs
        flow = sum(p[0] + p[1] for p in p_refs)
        o = jnp.dot(flow[:_N], wi_ref[...], preferred_element_type=jnp.float32)
        o = o + jnp.dot(flow[_N:], wo_ref[...], preferred_element_type=jnp.float32)
        o_ref[...] = jnp.maximum(o + b_ref[...], 0.0)

    full = lambda a: pl.BlockSpec(a.shape, lambda i: tuple(0 for _ in a.shape))
    return pl.pallas_call(
        body,
        grid=(1,),
        in_specs=[full(p) for p in partials_list] + [full(w_in), full(w_out), full(br)],
        out_specs=pl.BlockSpec((_N, w.shape[1]), lambda i: (0, 0)),
        out_shape=jax.ShapeDtypeStruct((_N, w.shape[1]), jnp.float32),
    )(*partials_list, w_in, w_out, br)


def _split_step_weights(params):
    (w1, b1), (w2, b2) = params['edge_model']
    (wi1, bi1), (wi2, bi2) = params['flow_in']
    (wo1, bo1), (wo2, bo2) = params['flow_out']
    (wc1, bc1), (wc2, bc2) = params['cls_edge']
    wio1 = jnp.concatenate([wi1, wo1], axis=1)
    bio1 = jnp.concatenate([bi1, bo1]).reshape(1, -1)
    wio2 = jnp.zeros((112, 64), jnp.float32)
    wio2 = wio2.at[:56, :32].set(wi2).at[56:, 32:].set(wo2)
    bio2 = jnp.concatenate([bi2, bo2]).reshape(1, -1)
    return (
        w1, b1.reshape(1, -1), w2, b2.reshape(1, -1),
        wio1, bio1, wio2, bio2,
        wc1, bc1.reshape(1, -1), wc2, bc2.reshape(1, -1),
    )


def kernel(x, edge_index, edge_attr, params):
    lat_n = _tc_mlp2(x, params['enc_node'], blk=2000)
    lat_e_full = _tc_mlp2(edge_attr, params['enc_edge'], blk=8000)
    wp = _split_step_weights(params)

    rows_ = edge_index[0]
    cols_ = edge_index[1]
    gid = edge_index.reshape(-1)
    idxp = jnp.where(rows_ < cols_, rows_ + _N, rows_)
    row3 = rows_.reshape(_E // _EB, 1, _EB)
    col3 = cols_.reshape(_E // _EB, 1, _EB)
    lat_e = lat_e_full
    init_e = lat_e_full

    outs = []
    for step in range(1, 5):
        g = _sc_gather(lat_n, gid)
        lat_e, v_msg, cls = _tc_dense_step(g, lat_e, init_e, row3, col3, wp)
        partials = _sc_scatter_add(v_msg, idxp, 2 * _N)
        lat_n = _tc_node_update([partials], params['node_update'])
        if step >= 2:
            outs.append(cls)
    return jnp.stack(outs)


# speedup vs baseline: 1.0691x; 1.0320x over previous
import functools

import jax
import jax.numpy as jnp
from jax import lax
from jax.experimental import pallas as pl
from jax.experimental.pallas import tpu as pltpu
from jax.experimental.pallas import tpu_sc as plsc

_N = 10000
_E = 320000
_NC, _NS = 2, 16
_W = 128

_EB = 1600
_NEB = _E // _EB


def _mesh():
    return plsc.VectorSubcoreMesh(
        core_axis_name="c", subcore_axis_name="s",
        num_cores=_NC, num_subcores=_NS)


_GB = 4


def _sc_gather(table, idx):
    b = idx.shape[0]
    d = table.shape[1]
    idx2 = idx.reshape(1, b)
    nblk = b // _W

    @functools.partial(
        pl.kernel,
        out_type=jax.ShapeDtypeStruct((b, d), table.dtype),
        mesh=_mesh(),
        compiler_params=pltpu.CompilerParams(use_tc_tiling_on_sc=False),
    )
    def k(t_hbm, i_hbm, o_hbm):
        def body(i_vmem, o_vmem):
            pltpu.sync_copy(t_hbm.at[i_vmem.at[0]], o_vmem)

        pltpu.emit_pipeline(
            body,
            grid=(nblk,),
            in_specs=[pl.BlockSpec((1, _W), lambda i: (0, i))],
            out_specs=[pl.BlockSpec((_W, d), lambda i: (i, 0))],
            core_axis_name=("c", "s"),
            dimension_semantics=(pltpu.PARALLEL,),
        )(i_hbm, o_hbm)

    return k(table, idx2)


def _sc_scatter_add(vals, idx, n_rows):
    e, d = vals.shape
    idx2 = idx.reshape(1, e)
    nblk = e // _W
    rps = n_rows // _NS
    zeros = jnp.zeros((n_rows, d), vals.dtype)

    @functools.partial(
        pl.kernel,
        out_type=jax.ShapeDtypeStruct((_NC, n_rows, d), vals.dtype),
        mesh=_mesh(),
        scratch_types=[pltpu.VMEM_SHARED((n_rows, d), vals.dtype)],
        compiler_params=pltpu.CompilerParams(use_tc_tiling_on_sc=False),
    )
    def k(v_hbm, i_hbm, z_hbm, o_hbm, acc):
        c = lax.axis_index("c")
        s = lax.axis_index("s")
        slab = pl.ds(s * rps, rps)
        pltpu.sync_copy(z_hbm.at[slab], acc.at[slab])
        plsc.subcore_barrier()

        def body(v_vmem, i_vmem):
            pltpu.sync_copy(v_vmem, acc.at[i_vmem.at[0]], add=True)

        pltpu.emit_pipeline(
            body,
            grid=(nblk,),
            in_specs=[pl.BlockSpec((_W, d), lambda i: (i, 0)),
                      pl.BlockSpec((1, _W), lambda i: (0, i))],
            out_specs=[],
            core_axis_name=("c", "s"),
            dimension_semantics=(pltpu.PARALLEL,),
        )(v_hbm, i_hbm)

        plsc.subcore_barrier()
        pltpu.sync_copy(acc.at[slab], o_hbm.at[c].at[slab])

    return k(vals, idx2, zeros)


def _tc_mlp2(xin, ps, blk, out_dtype=jnp.float32):
    (w1, b1), (w2, b2) = ps
    n = xin.shape[0]
    d_out = w2.shape[1]
    grid = n // blk
    b1r = b1.reshape(1, -1)
    b2r = b2.reshape(1, -1)

    def body(x_ref, w1_ref, b1_ref, w2_ref, b2_ref, o_ref):
        h = jnp.dot(x_ref[...], w1_ref[...], preferred_element_type=jnp.float32)
        h = jnp.maximum(h + b1_ref[...], 0.0)
        o = jnp.dot(h, w2_ref[...], preferred_element_type=jnp.float32)
        o_ref[...] = jnp.maximum(o + b2_ref[...], 0.0).astype(out_dtype)

    full = lambda a: pl.BlockSpec(a.shape, lambda i: (0, 0))
    return pl.pallas_call(
        body,
        grid=(grid,),
        in_specs=[
            pl.BlockSpec((blk, xin.shape[1]), lambda i: (i, 0)),
            full(w1), full(b1r), full(w2), full(b2r),
        ],
        out_specs=pl.BlockSpec((blk, d_out), lambda i: (i, 0)),
        out_shape=jax.ShapeDtypeStruct((n, d_out), out_dtype),
    )(xin, w1, b1r, w2, b2r)


def _tc_dense_step(g, lat_e, init_e, row3, col3, wp):
    (w1, b1, w2, b2, wio1, bio1, wio2, bio2, wc1, bc1, wc2, bc2) = wp
    ne = lat_e.shape[0]
    neb = ne // _EB

    def body(er, ec, gr, gc, le, ie,
             w1_, b1_, w2_, b2_, wio1_, bio1_, wio2_, bio2_,
             wc1_, bc1_, wc2_, bc2_,
             le_o, fc_o, cls_o):
        f32 = jnp.float32
        bf = jnp.bfloat16
        gc_b = gc[...].astype(bf)
        xcat = jnp.concatenate(
            [gr[...].astype(bf), gc_b, ie[...].astype(bf), le[...].astype(bf)],
            axis=1)
        h = jnp.dot(xcat, w1_[...], preferred_element_type=f32)
        h = jnp.maximum(h + b1_[...], 0.0)
        le_n = jnp.dot(h.astype(bf), w2_[...], preferred_element_type=f32)
        le_n = jnp.maximum(le_n + b2_[...], 0.0)
        le_o[...] = le_n

        nbr = jnp.concatenate([gc_b, le_n.astype(bf)], axis=1)
        hio = jnp.dot(nbr, wio1_[...], preferred_element_type=f32)
        hio = jnp.maximum(hio + bio1_[...], 0.0)
        f = jnp.dot(hio.astype(bf), wio2_[...], preferred_element_type=f32)
        f = jnp.maximum(f + bio2_[...], 0.0)

        rows = er[0, 0, :]
        cols = ec[0, 0, :]
        mi = (rows > cols).astype(f32)[:, None]
        mo = (rows < cols).astype(f32)[:, None]
        fc_o[...] = f[:, :32] * mi + f[:, 32:] * mo

        hc = jnp.maximum(
            jnp.dot(le_n, wc1_[...], preferred_element_type=f32) + bc1_[...], 0.0)
        cls_o[...] = jnp.dot(hc, wc2_[...], preferred_element_type=f32) + bc2_[...]

    full = lambda a: pl.BlockSpec(a.shape, lambda i: (0, 0))
    wspecs = [full(w) for w in wp]
    return pl.pallas_call(
        body,
        grid=(neb,),
        in_specs=[
            pl.BlockSpec((1, 1, _EB), lambda i: (i, 0, 0)),
            pl.BlockSpec((1, 1, _EB), lambda i: (i, 0, 0)),
            pl.BlockSpec((_EB, 32), lambda i: (i, 0)),
            pl.BlockSpec((_EB, 32), lambda i: (i + neb, 0)),
            pl.BlockSpec((_EB, 16), lambda i: (i, 0)),
            pl.BlockSpec((_EB, 16), lambda i: (i, 0)),
        ] + wspecs,
        out_specs=[
            pl.BlockSpec((_EB, 16), lambda i: (i, 0)),
            pl.BlockSpec((_EB, 32), lambda i: (i, 0)),
            pl.BlockSpec((_EB, 1), lambda i: (i, 0)),
        ],
        out_shape=[
            jax.ShapeDtypeStruct((ne, 16), jnp.float32),
            jax.ShapeDtypeStruct((ne, 32), jnp.float32),
            jax.ShapeDtypeStruct((ne, 1), jnp.float32),
        ],
    )(row3, col3, g, g, lat_e, init_e, *wp)


def _tc_node_update(partials_list, ps):
    (w, b) = ps[0]
    w_in, w_out = w[:32], w[32:]
    br = b.reshape(1, -1)

    def body(*refs):
        *p_refs, wi_ref, wo_ref, b_ref, o_ref = refs
        flow = sum(p[0] + p[1] for p in p_refs)
        o = jnp.dot(flow[:_N], wi_ref[...], preferred_element_type=jnp.float32)
        o = o + jnp.dot(flow[_N:], wo_ref[...], preferred_element_type=jnp.float32)
        o_ref[...] = jnp.maximum(o + b_ref[...], 0.0)

    full = lambda a: pl.BlockSpec(a.shape, lambda i: tuple(0 for _ in a.shape))
    return pl.pallas_call(
        body,
        grid=(1,),
        in_specs=[full(p) for p in partials_list] + [full(w_in), full(w_out), full(br)],
        out_specs=pl.BlockSpec((_N, w.shape[1]), lambda i: (0, 0)),
        out_shape=jax.ShapeDtypeStruct((_N, w.shape[1]), jnp.float32),
    )(*partials_list, w_in, w_out, br)


def _split_step_weights(params):
    (w1, b1), (w2, b2) = params['edge_model']
    (wi1, bi1), (wi2, bi2) = params['flow_in']
    (wo1, bo1), (wo2, bo2) = params['flow_out']
    (wc1, bc1), (wc2, bc2) = params['cls_edge']
    wio1 = jnp.concatenate([wi1, wo1], axis=1)
    bio1 = jnp.concatenate([bi1, bo1]).reshape(1, -1)
    wio2 = jnp.zeros((112, 64), jnp.float32)
    wio2 = wio2.at[:56, :32].set(wi2).at[56:, 32:].set(wo2)
    bio2 = jnp.concatenate([bi2, bo2]).reshape(1, -1)
    bf = jnp.bfloat16
    return (
        w1.astype(bf), b1.reshape(1, -1), w2.astype(bf), b2.reshape(1, -1),
        wio1.astype(bf), bio1, wio2.astype(bf), bio2,
        wc1, bc1.reshape(1, -1), wc2, bc2.reshape(1, -1),
    )


def kernel(x, edge_index, edge_attr, params):
    lat_n = _tc_mlp2(x, params['enc_node'], blk=2000)
    lat_e_full = _tc_mlp2(edge_attr, params['enc_edge'], blk=8000)
    wp = _split_step_weights(params)

    rows_ = edge_index[0]
    cols_ = edge_index[1]
    gid = edge_index.reshape(-1)
    idxp = jnp.where(rows_ < cols_, rows_ + _N, rows_)
    row3 = rows_.reshape(_E // _EB, 1, _EB)
    col3 = cols_.reshape(_E // _EB, 1, _EB)
    lat_e = lat_e_full
    init_e = lat_e_full

    outs = []
    for step in range(1, 5):
        g = _sc_gather(lat_n, gid)
        lat_e, v_msg, cls = _tc_dense_step(g, lat_e, init_e, row3, col3, wp)
        partials = _sc_scatter_add(v_msg, idxp, 2 * _N)
        lat_n = _tc_node_update([partials], params['node_update'])
        if step >= 2:
            outs.append(cls)
    return jnp.stack(outs)


# speedup vs baseline: 1.1240x; 1.0513x over previous
import functools

import jax
import jax.numpy as jnp
from jax import lax
from jax.experimental import pallas as pl
from jax.experimental.pallas import tpu as pltpu
from jax.experimental.pallas import tpu_sc as plsc

_N = 10000
_E = 320000
_NC, _NS = 2, 16
_W = 128

_EB = 3200
_NEB = _E // _EB


def _mesh():
    return plsc.VectorSubcoreMesh(
        core_axis_name="c", subcore_axis_name="s",
        num_cores=_NC, num_subcores=_NS)


_GB = 4


def _sc_gather(table, idx):
    b = idx.shape[0]
    d = table.shape[1]
    idx2 = idx.reshape(1, b)
    nblk = b // _W

    @functools.partial(
        pl.kernel,
        out_type=jax.ShapeDtypeStruct((b, d), table.dtype),
        mesh=_mesh(),
        compiler_params=pltpu.CompilerParams(use_tc_tiling_on_sc=False),
    )
    def k(t_hbm, i_hbm, o_hbm):
        def body(i_vmem, o_vmem):
            pltpu.sync_copy(t_hbm.at[i_vmem.at[0]], o_vmem)

        pltpu.emit_pipeline(
            body,
            grid=(nblk,),
            in_specs=[pl.BlockSpec((1, _W), lambda i: (0, i))],
            out_specs=[pl.BlockSpec((_W, d), lambda i: (i, 0))],
            core_axis_name=("c", "s"),
            dimension_semantics=(pltpu.PARALLEL,),
        )(i_hbm, o_hbm)

    return k(table, idx2)


def _sc_scatter_add(vals, idx, n_rows):
    e, d = vals.shape
    idx2 = idx.reshape(1, e)
    nblk = e // _W
    rps = n_rows // _NS
    zeros = jnp.zeros((n_rows, d), vals.dtype)

    @functools.partial(
        pl.kernel,
        out_type=jax.ShapeDtypeStruct((_NC, n_rows, d), vals.dtype),
        mesh=_mesh(),
        scratch_types=[pltpu.VMEM_SHARED((n_rows, d), vals.dtype)],
        compiler_params=pltpu.CompilerParams(use_tc_tiling_on_sc=False),
    )
    def k(v_hbm, i_hbm, z_hbm, o_hbm, acc):
        c = lax.axis_index("c")
        s = lax.axis_index("s")
        slab = pl.ds(s * rps, rps)
        pltpu.sync_copy(z_hbm.at[slab], acc.at[slab])
        plsc.subcore_barrier()

        def body(v_vmem, i_vmem):
            pltpu.sync_copy(v_vmem, acc.at[i_vmem.at[0]], add=True)

        pltpu.emit_pipeline(
            body,
            grid=(nblk,),
            in_specs=[pl.BlockSpec((_W, d), lambda i: (i, 0)),
                      pl.BlockSpec((1, _W), lambda i: (0, i))],
            out_specs=[],
            core_axis_name=("c", "s"),
            dimension_semantics=(pltpu.PARALLEL,),
        )(v_hbm, i_hbm)

        plsc.subcore_barrier()
        pltpu.sync_copy(acc.at[slab], o_hbm.at[c].at[slab])

    return k(vals, idx2, zeros)


def _tc_mlp2(xin, ps, blk, out_dtype=jnp.float32):
    (w1, b1), (w2, b2) = ps
    n = xin.shape[0]
    d_out = w2.shape[1]
    grid = n // blk
    b1r = b1.reshape(1, -1)
    b2r = b2.reshape(1, -1)

    def body(x_ref, w1_ref, b1_ref, w2_ref, b2_ref, o_ref):
        h = jnp.dot(x_ref[...], w1_ref[...], preferred_element_type=jnp.float32)
        h = jnp.maximum(h + b1_ref[...], 0.0)
        o = jnp.dot(h, w2_ref[...], preferred_element_type=jnp.float32)
        o_ref[...] = jnp.maximum(o + b2_ref[...], 0.0).astype(out_dtype)

    full = lambda a: pl.BlockSpec(a.shape, lambda i: (0, 0))
    return pl.pallas_call(
        body,
        grid=(grid,),
        in_specs=[
            pl.BlockSpec((blk, xin.shape[1]), lambda i: (i, 0)),
            full(w1), full(b1r), full(w2), full(b2r),
        ],
        out_specs=pl.BlockSpec((blk, d_out), lambda i: (i, 0)),
        out_shape=jax.ShapeDtypeStruct((n, d_out), out_dtype),
    )(xin, w1, b1r, w2, b2r)


def _tc_dense_step(g, lat_e, init_e, row3, col3, wp):
    (w1, b1, w2, b2, wio1, bio1, wio2, bio2, wc1, bc1, wc2, bc2) = wp
    ne = lat_e.shape[0]
    neb = ne // _EB

    def body(er, ec, gr, gc, le, ie,
             w1_, b1_, w2_, b2_, wio1_, bio1_, wio2_, bio2_,
             wc1_, bc1_, wc2_, bc2_,
             le_o, fc_o, cls_o):
        f32 = jnp.float32
        bf = jnp.bfloat16
        gc_b = gc[...].astype(bf)
        xcat = jnp.concatenate(
            [gr[...].astype(bf), gc_b, ie[...].astype(bf), le[...].astype(bf)],
            axis=1)
        h = jnp.dot(xcat, w1_[...], preferred_element_type=f32)
        h = jnp.maximum(h + b1_[...], 0.0)
        le_n = jnp.dot(h.astype(bf), w2_[...], preferred_element_type=f32)
        le_n = jnp.maximum(le_n + b2_[...], 0.0)
        le_o[...] = le_n

        nbr = jnp.concatenate([gc_b, le_n.astype(bf)], axis=1)
        hio = jnp.dot(nbr, wio1_[...], preferred_element_type=f32)
        hio = jnp.maximum(hio + bio1_[...], 0.0)
        f = jnp.dot(hio.astype(bf), wio2_[...], preferred_element_type=f32)
        f = jnp.maximum(f + bio2_[...], 0.0)

        rows = er[0, 0, :]
        cols = ec[0, 0, :]
        mi = (rows > cols).astype(f32)[:, None]
        mo = (rows < cols).astype(f32)[:, None]
        fc_o[...] = f[:, :32] * mi + f[:, 32:] * mo

        hc = jnp.maximum(
            jnp.dot(le_n, wc1_[...], preferred_element_type=f32) + bc1_[...], 0.0)
        cls_o[...] = jnp.dot(hc, wc2_[...], preferred_element_type=f32) + bc2_[...]

    full = lambda a: pl.BlockSpec(a.shape, lambda i: (0, 0))
    wspecs = [full(w) for w in wp]
    return pl.pallas_call(
        body,
        grid=(neb,),
        in_specs=[
            pl.BlockSpec((1, 1, _EB), lambda i: (i, 0, 0)),
            pl.BlockSpec((1, 1, _EB), lambda i: (i, 0, 0)),
            pl.BlockSpec((_EB, 32), lambda i: (i, 0)),
            pl.BlockSpec((_EB, 32), lambda i: (i + neb, 0)),
            pl.BlockSpec((_EB, 16), lambda i: (i, 0)),
            pl.BlockSpec((_EB, 16), lambda i: (i, 0)),
        ] + wspecs,
        out_specs=[
            pl.BlockSpec((_EB, 16), lambda i: (i, 0)),
            pl.BlockSpec((_EB, 32), lambda i: (i, 0)),
            pl.BlockSpec((_EB, 1), lambda i: (i, 0)),
        ],
        out_shape=[
            jax.ShapeDtypeStruct((ne, 16), jnp.float32),
            jax.ShapeDtypeStruct((ne, 32), jnp.float32),
            jax.ShapeDtypeStruct((ne, 1), jnp.float32),
        ],
    )(row3, col3, g, g, lat_e, init_e, *wp)


def _tc_node_update(partials_list, ps):
    (w, b) = ps[0]
    w_in, w_out = w[:32], w[32:]
    br = b.reshape(1, -1)

    def body(*refs):
        *p_refs, wi_ref, wo_ref, b_ref, o_ref = refs
        flow = sum(p[0] + p[1] for p in p_refs)
        o = jnp.dot(flow[:_N], wi_ref[...], preferred_element_type=jnp.float32)
        o = o + jnp.dot(flow[_N:], wo_ref[...], preferred_element_type=jnp.float32)
        o_ref[...] = jnp.maximum(o + b_ref[...], 0.0)

    full = lambda a: pl.BlockSpec(a.shape, lambda i: tuple(0 for _ in a.shape))
    return pl.pallas_call(
        body,
        grid=(1,),
        in_specs=[full(p) for p in partials_list] + [full(w_in), full(w_out), full(br)],
        out_specs=pl.BlockSpec((_N, w.shape[1]), lambda i: (0, 0)),
        out_shape=jax.ShapeDtypeStruct((_N, w.shape[1]), jnp.float32),
    )(*partials_list, w_in, w_out, br)


def _split_step_weights(params):
    (w1, b1), (w2, b2) = params['edge_model']
    (wi1, bi1), (wi2, bi2) = params['flow_in']
    (wo1, bo1), (wo2, bo2) = params['flow_out']
    (wc1, bc1), (wc2, bc2) = params['cls_edge']
    wio1 = jnp.concatenate([wi1, wo1], axis=1)
    bio1 = jnp.concatenate([bi1, bo1]).reshape(1, -1)
    wio2 = jnp.zeros((112, 64), jnp.float32)
    wio2 = wio2.at[:56, :32].set(wi2).at[56:, 32:].set(wo2)
    bio2 = jnp.concatenate([bi2, bo2]).reshape(1, -1)
    bf = jnp.bfloat16
    return (
        w1.astype(bf), b1.reshape(1, -1), w2.astype(bf), b2.reshape(1, -1),
        wio1.astype(bf), bio1, wio2.astype(bf), bio2,
        wc1, bc1.reshape(1, -1), wc2, bc2.reshape(1, -1),
    )


def kernel(x, edge_index, edge_attr, params):
    lat_n = _tc_mlp2(x, params['enc_node'], blk=2000)
    lat_e_full = _tc_mlp2(edge_attr, params['enc_edge'], blk=8000)
    wp = _split_step_weights(params)

    rows_ = edge_index[0]
    cols_ = edge_index[1]
    gid = edge_index.reshape(-1)
    idxp = jnp.where(rows_ < cols_, rows_ + _N, rows_)
    row3 = rows_.reshape(_E // _EB, 1, _EB)
    col3 = cols_.reshape(_E // _EB, 1, _EB)
    lat_e = lat_e_full
    init_e = lat_e_full

    outs = []
    for step in range(1, 5):
        g = _sc_gather(lat_n, gid)
        lat_e, v_msg, cls = _tc_dense_step(g, lat_e, init_e, row3, col3, wp)
        partials = _sc_scatter_add(v_msg, idxp, 2 * _N)
        lat_n = _tc_node_update([partials], params['node_update'])
        if step >= 2:
            outs.append(cls)
    return jnp.stack(outs)


# speedup vs baseline: 1.1491x; 1.0223x over previous
import functools

import jax
import jax.numpy as jnp
from jax import lax
from jax.experimental import pallas as pl
from jax.experimental.pallas import tpu as pltpu
from jax.experimental.pallas import tpu_sc as plsc

_N = 10000
_E = 320000
_NC, _NS = 2, 16
_W = 128

_EB = 6400
_NEB = _E // _EB


def _mesh():
    return plsc.VectorSubcoreMesh(
        core_axis_name="c", subcore_axis_name="s",
        num_cores=_NC, num_subcores=_NS)


_GB = 4


def _sc_gather(table, idx):
    b = idx.shape[0]
    d = table.shape[1]
    idx2 = idx.reshape(1, b)
    nblk = b // _W

    @functools.partial(
        pl.kernel,
        out_type=jax.ShapeDtypeStruct((b, d), table.dtype),
        mesh=_mesh(),
        compiler_params=pltpu.CompilerParams(use_tc_tiling_on_sc=False),
    )
    def k(t_hbm, i_hbm, o_hbm):
        def body(i_vmem, o_vmem):
            pltpu.sync_copy(t_hbm.at[i_vmem.at[0]], o_vmem)

        pltpu.emit_pipeline(
            body,
            grid=(nblk,),
            in_specs=[pl.BlockSpec((1, _W), lambda i: (0, i))],
            out_specs=[pl.BlockSpec((_W, d), lambda i: (i, 0))],
            core_axis_name=("c", "s"),
            dimension_semantics=(pltpu.PARALLEL,),
        )(i_hbm, o_hbm)

    return k(table, idx2)


def _sc_scatter_add(vals, idx, n_rows):
    e, d = vals.shape
    idx2 = idx.reshape(1, e)
    nblk = e // _W
    rps = n_rows // _NS
    zeros = jnp.zeros((n_rows, d), vals.dtype)

    @functools.partial(
        pl.kernel,
        out_type=jax.ShapeDtypeStruct((_NC, n_rows, d), vals.dtype),
        mesh=_mesh(),
        scratch_types=[pltpu.VMEM_SHARED((n_rows, d), vals.dtype)],
        compiler_params=pltpu.CompilerParams(use_tc_tiling_on_sc=False),
    )
    def k(v_hbm, i_hbm, z_hbm, o_hbm, acc):
        c = lax.axis_index("c")
        s = lax.axis_index("s")
        slab = pl.ds(s * rps, rps)
        pltpu.sync_copy(z_hbm.at[slab], acc.at[slab])
        plsc.subcore_barrier()

        def body(v_vmem, i_vmem):
            pltpu.sync_copy(v_vmem, acc.at[i_vmem.at[0]], add=True)

        pltpu.emit_pipeline(
            body,
            grid=(nblk,),
            in_specs=[pl.BlockSpec((_W, d), lambda i: (i, 0)),
                      pl.BlockSpec((1, _W), lambda i: (0, i))],
            out_specs=[],
            core_axis_name=("c", "s"),
            dimension_semantics=(pltpu.PARALLEL,),
        )(v_hbm, i_hbm)

        plsc.subcore_barrier()
        pltpu.sync_copy(acc.at[slab], o_hbm.at[c].at[slab])

    return k(vals, idx2, zeros)


def _tc_mlp2(xin, ps, blk, out_dtype=jnp.float32):
    (w1, b1), (w2, b2) = ps
    n = xin.shape[0]
    d_out = w2.shape[1]
    grid = n // blk
    b1r = b1.reshape(1, -1)
    b2r = b2.reshape(1, -1)

    def body(x_ref, w1_ref, b1_ref, w2_ref, b2_ref, o_ref):
        h = jnp.dot(x_ref[...], w1_ref[...], preferred_element_type=jnp.float32)
        h = jnp.maximum(h + b1_ref[...], 0.0)
        o = jnp.dot(h, w2_ref[...], preferred_element_type=jnp.float32)
        o_ref[...] = jnp.maximum(o + b2_ref[...], 0.0).astype(out_dtype)

    full = lambda a: pl.BlockSpec(a.shape, lambda i: (0, 0))
    return pl.pallas_call(
        body,
        grid=(grid,),
        in_specs=[
            pl.BlockSpec((blk, xin.shape[1]), lambda i: (i, 0)),
            full(w1), full(b1r), full(w2), full(b2r),
        ],
        out_specs=pl.BlockSpec((blk, d_out), lambda i: (i, 0)),
        out_shape=jax.ShapeDtypeStruct((n, d_out), out_dtype),
    )(xin, w1, b1r, w2, b2r)


def _tc_dense_step(g, lat_e, init_e, row3, col3, wp):
    (w1, b1, w2, b2, wio1, bio1, wio2, bio2, wc1, bc1, wc2, bc2) = wp
    ne = lat_e.shape[0]
    neb = ne // _EB

    def body(er, ec, gr, gc, le, ie,
             w1_, b1_, w2_, b2_, wio1_, bio1_, wio2_, bio2_,
             wc1_, bc1_, wc2_, bc2_,
             le_o, fc_o, cls_o):
        f32 = jnp.float32
        bf = jnp.bfloat16
        gc_b = gc[...].astype(bf)
        xcat = jnp.concatenate(
            [gr[...].astype(bf), gc_b, ie[...].astype(bf), le[...].astype(bf)],
            axis=1)
        h = jnp.dot(xcat, w1_[...], preferred_element_type=f32)
        h = jnp.maximum(h + b1_[...], 0.0)
        le_n = jnp.dot(h.astype(bf), w2_[...], preferred_element_type=f32)
        le_n = jnp.maximum(le_n + b2_[...], 0.0)
        le_o[...] = le_n

        nbr = jnp.concatenate([gc_b, le_n.astype(bf)], axis=1)
        hio = jnp.dot(nbr, wio1_[...], preferred_element_type=f32)
        hio = jnp.maximum(hio + bio1_[...], 0.0)
        f = jnp.dot(hio.astype(bf), wio2_[...], preferred_element_type=f32)
        f = jnp.maximum(f + bio2_[...], 0.0)

        rows = er[0, 0, :]
        cols = ec[0, 0, :]
        mi = (rows > cols).astype(f32)[:, None]
        mo = (rows < cols).astype(f32)[:, None]
        fc_o[...] = f[:, :32] * mi + f[:, 32:] * mo

        hc = jnp.maximum(
            jnp.dot(le_n, wc1_[...], preferred_element_type=f32) + bc1_[...], 0.0)
        cls_o[...] = jnp.dot(hc, wc2_[...], preferred_element_type=f32) + bc2_[...]

    full = lambda a: pl.BlockSpec(a.shape, lambda i: (0, 0))
    wspecs = [full(w) for w in wp]
    return pl.pallas_call(
        body,
        grid=(neb,),
        in_specs=[
            pl.BlockSpec((1, 1, _EB), lambda i: (i, 0, 0)),
            pl.BlockSpec((1, 1, _EB), lambda i: (i, 0, 0)),
            pl.BlockSpec((_EB, 32), lambda i: (i, 0)),
            pl.BlockSpec((_EB, 32), lambda i: (i + neb, 0)),
            pl.BlockSpec((_EB, 16), lambda i: (i, 0)),
            pl.BlockSpec((_EB, 16), lambda i: (i, 0)),
        ] + wspecs,
        out_specs=[
            pl.BlockSpec((_EB, 16), lambda i: (i, 0)),
            pl.BlockSpec((_EB, 32), lambda i: (i, 0)),
            pl.BlockSpec((_EB, 1), lambda i: (i, 0)),
        ],
        out_shape=[
            jax.ShapeDtypeStruct((ne, 16), jnp.float32),
            jax.ShapeDtypeStruct((ne, 32), jnp.float32),
            jax.ShapeDtypeStruct((ne, 1), jnp.float32),
        ],
    )(row3, col3, g, g, lat_e, init_e, *wp)


def _tc_node_update(partials_list, ps):
    (w, b) = ps[0]
    w_in, w_out = w[:32], w[32:]
    br = b.reshape(1, -1)

    def body(*refs):
        *p_refs, wi_ref, wo_ref, b_ref, o_ref = refs
        flow = sum(p[0] + p[1] for p in p_refs)
        o = jnp.dot(flow[:_N], wi_ref[...], preferred_element_type=jnp.float32)
        o = o + jnp.dot(flow[_N:], wo_ref[...], preferred_element_type=jnp.float32)
        o_ref[...] = jnp.maximum(o + b_ref[...], 0.0)

    full = lambda a: pl.BlockSpec(a.shape, lambda i: tuple(0 for _ in a.shape))
    return pl.pallas_call(
        body,
        grid=(1,),
        in_specs=[full(p) for p in partials_list] + [full(w_in), full(w_out), full(br)],
        out_specs=pl.BlockSpec((_N, w.shape[1]), lambda i: (0, 0)),
        out_shape=jax.ShapeDtypeStruct((_N, w.shape[1]), jnp.float32),
    )(*partials_list, w_in, w_out, br)


def _split_step_weights(params):
    (w1, b1), (w2, b2) = params['edge_model']
    (wi1, bi1), (wi2, bi2) = params['flow_in']
    (wo1, bo1), (wo2, bo2) = params['flow_out']
    (wc1, bc1), (wc2, bc2) = params['cls_edge']
    wio1 = jnp.concatenate([wi1, wo1], axis=1)
    bio1 = jnp.concatenate([bi1, bo1]).reshape(1, -1)
    wio2 = jnp.zeros((112, 64), jnp.float32)
    wio2 = wio2.at[:56, :32].set(wi2).at[56:, 32:].set(wo2)
    bio2 = jnp.concatenate([bi2, bo2]).reshape(1, -1)
    bf = jnp.bfloat16
    return (
        w1.astype(bf), b1.reshape(1, -1), w2.astype(bf), b2.reshape(1, -1),
        wio1.astype(bf), bio1, wio2.astype(bf), bio2,
        wc1, bc1.reshape(1, -1), wc2, bc2.reshape(1, -1),
    )


def kernel(x, edge_index, edge_attr, params):
    lat_n = _tc_mlp2(x, params['enc_node'], blk=2000)
    lat_e_full = _tc_mlp2(edge_attr, params['enc_edge'], blk=8000)
    wp = _split_step_weights(params)

    rows_ = edge_index[0]
    cols_ = edge_index[1]
    gid = edge_index.reshape(-1)
    idxp = jnp.where(rows_ < cols_, rows_ + _N, rows_)
    row3 = rows_.reshape(_E // _EB, 1, _EB)
    col3 = cols_.reshape(_E // _EB, 1, _EB)
    lat_e = lat_e_full
    init_e = lat_e_full

    outs = []
    for step in range(1, 5):
        g = _sc_gather(lat_n, gid)
        lat_e, v_msg, cls = _tc_dense_step(g, lat_e, init_e, row3, col3, wp)
        partials = _sc_scatter_add(v_msg, idxp, 2 * _N)
        lat_n = _tc_node_update([partials], params['node_update'])
        if step >= 2:
            outs.append(cls)
    return jnp.stack(outs)
